# Initial kernel scaffold; baseline (speedup 1.0000x reference)
import jax
import jax.numpy as jnp
from jax.experimental import pallas as pl


def kernel(xs_0, xs_1, ess_0, ess_1, enc_al, emb_test, Wt, bt, Ws, bs, Eself, Wd, bd):
    raise NotImplementedError("write your pallas kernel here")


# speedup vs baseline: 2.6239x; 2.6239x over previous
import functools

import jax
import jax.numpy as jnp
from jax import lax
from jax.experimental import pallas as pl
from jax.experimental.pallas import tpu as pltpu
from jax.experimental.pallas import tpu_sc as plsc

N = 10000
D = 128
E = 160000
NW = 32
SZ = 320
NP = NW * SZ
F = 8192
LCAP = 20 * F
CH = 3200
NEG_INF = float("-inf")

_MESH = plsc.VectorSubcoreMesh(core_axis_name="c", subcore_axis_name="s")
_SC_PARAMS = pltpu.CompilerParams(needs_layout_passes=False)


def _worker_id():
    return lax.axis_index("s") * 2 + lax.axis_index("c")


@functools.partial(
    pl.kernel,
    mesh=_MESH,
    compiler_params=_SC_PARAMS,
    out_type=[
        jax.ShapeDtypeStruct((NP, D), jnp.float32),
        jax.ShapeDtypeStruct((NW * LCAP,), jnp.int32),
        jax.ShapeDtypeStruct((NW * 16,), jnp.int32),
        jax.ShapeDtypeStruct((NW * SZ * 16,), jnp.int32),
        jax.ShapeDtypeStruct((NW * LCAP,), jnp.int32),
        jax.ShapeDtypeStruct((NW * 16,), jnp.int32),
        jax.ShapeDtypeStruct((NW * SZ * 16,), jnp.int32),
    ],
    scratch_types=[
        pltpu.VMEM((SZ,), jnp.int32),
        pltpu.VMEM((SZ, D), jnp.float32),
        pltpu.VMEM((CH,), jnp.int32),
        pltpu.VMEM((CH,), jnp.int32),
        pltpu.VMEM((F + 16,), jnp.int32),
        pltpu.VMEM((SZ * 16,), jnp.int32),
        pltpu.VMEM((16,), jnp.int32),
        pltpu.SemaphoreType.DMA,
    ],
)
def _sc_prep(enc_hbm, xs0_hbm, ess0_hbm, ess1_hbm,
             x0_hbm, lists0_hbm, cnts0_hbm, self0_hbm,
             lists1_hbm, cnts1_hbm, self1_hbm,
             idxv, rows, srcv, dstv, buf, self16, cnt16, sem):
    w = _worker_id()
    lo = w * SZ
    iota = lax.iota(jnp.int32, 16)
    ones = jnp.ones((16,), jnp.int32)
    zeros = jnp.zeros((16,), jnp.int32)

    pltpu.sync_copy(xs0_hbm.at[pl.ds(pl.multiple_of(lo, SZ), SZ)], idxv)
    for kb in range(SZ // 64):
        pltpu.async_copy(
            enc_hbm.at[idxv.at[pl.ds(kb * 64, 64)]],
            rows.at[pl.ds(kb * 64, 64)], sem).wait()
    pltpu.sync_copy(rows, x0_hbm.at[pl.ds(pl.multiple_of(lo, SZ), SZ)])

    for ess_hbm, lists_hbm, cnts_hbm, selfo_hbm in (
            (ess0_hbm, lists0_hbm, cnts0_hbm, self0_hbm),
            (ess1_hbm, lists1_hbm, cnts1_hbm, self1_hbm)):

        def zbody(i, _):
            self16[pl.ds(i * 16, 16)] = zeros
            return 0
        lax.fori_loop(0, SZ, zbody, 0)

        def chunk_body(ci, carry):
            pltpu.sync_copy(ess_hbm.at[0, pl.ds(ci * CH, CH)], srcv)
            pltpu.sync_copy(ess_hbm.at[1, pl.ds(ci * CH, CH)], dstv)

            def vreg_body(i, c2):
                off, gh = c2
                s = srcv[pl.ds(i * 16, 16)]
                d = dstv[pl.ds(i * 16, 16)]
                inr = (d >= lo) & (d < lo + SZ)
                dl = d - lo
                selfm = inr & (s == d)
                nons = inr & (s != d)
                plsc.store_scatter(self16, [dl * 16 + iota], ones, mask=selfm)
                pack = s | (dl << 16)
                plsc.store_compressed(buf.at[pl.ds(off, 16)], pack, mask=nons)
                noff = off + plsc.all_reduce_population_count(nons)[0]

                def do_flush(c3):
                    o, g = c3
                    pltpu.sync_copy(buf.at[pl.ds(0, F)],
                                    lists_hbm.at[pl.ds(pl.multiple_of(w * LCAP + g, F), F)])
                    tail = buf[pl.ds(F, 16)]
                    buf[pl.ds(0, 16)] = tail
                    return o - F, g + F

                return lax.cond(noff >= F, do_flush, lambda c3: c3, (noff, gh))

            return lax.fori_loop(0, CH // 16, vreg_body, carry)

        off, gh = lax.fori_loop(0, E // CH, chunk_body,
                                (jnp.int32(0), jnp.int32(0)))
        pltpu.sync_copy(buf.at[pl.ds(0, F)],
                        lists_hbm.at[pl.ds(pl.multiple_of(w * LCAP + gh, F), F)])
        cnt16[pl.ds(0, 16)] = jnp.full((16,), gh + off, jnp.int32)
        pltpu.sync_copy(cnt16, cnts_hbm.at[pl.ds(pl.multiple_of(w * 16, 16), 16)])
        pltpu.sync_copy(self16, selfo_hbm.at[pl.ds(pl.multiple_of(w * SZ * 16, SZ * 16), SZ * 16)])


def _etype_pass(w, lo, lists_hbm, cnts_hbm, selfi_hbm, hs_hbm, ht_hbm, ce_hbm,
                out_hbm, acc, rows, lchunk, srcs, dstl, self16, cnt16, cev,
                htb, hsb, xob, sem):
    ninf16 = jnp.full((16,), NEG_INF, jnp.float32)

    def init_body(dd, _):
        for q in range(8):
            acc[dd, pl.ds(16 * q, 16)] = ninf16
        return 0
    lax.fori_loop(0, SZ, init_body, 0)

    pltpu.sync_copy(cnts_hbm.at[pl.ds(pl.multiple_of(w * 16, 16), 16)], cnt16)
    pltpu.sync_copy(selfi_hbm.at[pl.ds(pl.multiple_of(w * SZ * 16, SZ * 16), SZ * 16)], self16)
    pltpu.sync_copy(ce_hbm, cev)
    cnt = cnt16[pl.ds(0, 16)][0]

    nch = (cnt + 1023) // 1024

    def chunk_body(ci, _):
        cbase = ci * 1024
        pltpu.sync_copy(lists_hbm.at[pl.ds(pl.multiple_of(w * LCAP + cbase, 1024), 1024)], lchunk)

        def unpack_body(i, _):
            p = lchunk[pl.ds(i * 16, 16)]
            lane = cbase + i * 16 + lax.iota(jnp.int32, 16)
            valid = lane < cnt
            s = jnp.where(valid, p & 0xFFFF, 0)
            srcs[pl.ds(i * 16, 16)] = s
            dstl[pl.ds(i * 16, 16)] = p >> 16
            return 0
        lax.fori_loop(0, 64, unpack_body, 0)

        for kb in range(8):
            rem = cnt - cbase - kb * 128

            @pl.when(rem > 0)
            def _():
                pltpu.async_copy(
                    hs_hbm.at[srcs.at[pl.ds(kb * 128, 128)]], rows, sem).wait()

                def edge_body(e, _):
                    dd = dstl[pl.ds(kb * 128 + e, 16)][0]
                    for q in range(8):
                        a = acc[dd, pl.ds(16 * q, 16)]
                        g = rows[e, pl.ds(16 * q, 16)]
                        acc[dd, pl.ds(16 * q, 16)] = jnp.maximum(a, g)
                    return 0
                lax.fori_loop(0, jnp.minimum(rem, 128), edge_body, 0)
        return 0

    lax.fori_loop(0, nch, chunk_body, 0)

    for blk in range(SZ // 64):
        base = blk * 64
        pltpu.sync_copy(ht_hbm.at[pl.ds(pl.multiple_of(lo + base, 64), 64)], htb)
        pltpu.sync_copy(hs_hbm.at[pl.ds(pl.multiple_of(lo + base, 64), 64)], hsb)

        def comb_body(dd, _):
            flag = self16[pl.ds((base + dd) * 16, 16)]
            has_self = lax.reduce_max(flag, axes=(0,))
            pen = jnp.where(has_self > 0, jnp.float32(0), jnp.float32(NEG_INF))
            for q in range(8):
                sl = pl.ds(16 * q, 16)
                cand0 = acc[base + dd, sl] + cev[1, sl]
                cand1 = hsb[dd, sl] + cev[2, sl] + pen
                agg = htb[dd, sl] + cev[0, sl] + jnp.maximum(cand0, cand1)
                xob[dd, sl] = jnp.maximum(agg, jnp.float32(0))
            return 0
        lax.fori_loop(0, 64, comb_body, 0)
        pltpu.sync_copy(xob, out_hbm.at[pl.ds(pl.multiple_of(lo + base, 64), 64)])


def _make_layer_kernel(both):
    n_out = 2 if both else 1
    scratch = [
        pltpu.VMEM((SZ, D), jnp.float32),
        pltpu.VMEM((128, D), jnp.float32),
        pltpu.VMEM((1024,), jnp.int32),
        pltpu.VMEM((1024 + 16,), jnp.int32),
        pltpu.VMEM((1024 + 16,), jnp.int32),
        pltpu.VMEM((SZ * 16,), jnp.int32),
        pltpu.VMEM((16,), jnp.int32),
        pltpu.VMEM((3, D), jnp.float32),
        pltpu.VMEM((64, D), jnp.float32),
        pltpu.VMEM((64, D), jnp.float32),
        pltpu.VMEM((64, D), jnp.float32),
        pltpu.SemaphoreType.DMA,
    ]

    if both:
        def body(l0, c0, s0, hsa, hta, ce0, l1, c1, s1, hsb_, htb_, ce1,
                 x1new, x0new, *sc):
            w = _worker_id()
            lo = w * SZ
            _etype_pass(w, lo, l0, c0, s0, hsa, hta, ce0, x1new, *sc)
            _etype_pass(w, lo, l1, c1, s1, hsb_, htb_, ce1, x0new, *sc)
    else:
        def body(l1, c1, s1, hsb_, htb_, ce1, x0new, *sc):
            w = _worker_id()
            lo = w * SZ
            _etype_pass(w, lo, l1, c1, s1, hsb_, htb_, ce1, x0new, *sc)

    return functools.partial(
        pl.kernel,
        mesh=_MESH,
        compiler_params=_SC_PARAMS,
        out_type=[jax.ShapeDtypeStruct((NP, D), jnp.float32)] * n_out,
        scratch_types=scratch,
    )(body)


_sc_layer_both = _make_layer_kernel(True)
_sc_layer_last = _make_layer_kernel(False)


_BM = 1280


def _tc_mm(x0, x1, w4):
    def body(x0_ref, x1_ref, w_ref, h0, h1, h2, h3):
        a = x0_ref[...]
        b = x1_ref[...]
        h0[...] = jnp.dot(a, w_ref[0], preferred_element_type=jnp.float32)
        h1[...] = jnp.dot(a, w_ref[1], preferred_element_type=jnp.float32)
        h2[...] = jnp.dot(b, w_ref[2], preferred_element_type=jnp.float32)
        h3[...] = jnp.dot(b, w_ref[3], preferred_element_type=jnp.float32)

    return pl.pallas_call(
        body,
        grid=(NP // _BM,),
        in_specs=[
            pl.BlockSpec((_BM, D), lambda i: (i, 0)),
            pl.BlockSpec((_BM, D), lambda i: (i, 0)),
            pl.BlockSpec((4, D, D), lambda i: (0, 0, 0)),
        ],
        out_specs=[pl.BlockSpec((_BM, D), lambda i: (i, 0))] * 4,
        out_shape=[jax.ShapeDtypeStruct((NP, D), jnp.float32)] * 4,
    )(x0, x1, w4)


def _tc_head(x0, wd, bd):
    def body(x_ref, wd_ref, bd_ref, last_ref, p_ref):
        t = jnp.dot(x_ref[...], wd_ref[...],
                    preferred_element_type=jnp.float32) + bd_ref[...]
        last_ref[...] = t
        m = jnp.max(t, axis=1, keepdims=True)
        e = jnp.exp(t - m)
        p_ref[...] = e / jnp.sum(e, axis=1, keepdims=True)

    return pl.pallas_call(
        body,
        grid=(NP // _BM,),
        in_specs=[
            pl.BlockSpec((_BM, D), lambda i: (i, 0)),
            pl.BlockSpec((D, 3), lambda i: (0, 0)),
            pl.BlockSpec((1, 3), lambda i: (0, 0)),
        ],
        out_specs=[pl.BlockSpec((_BM, 3), lambda i: (i, 0))] * 2,
        out_shape=[jax.ShapeDtypeStruct((NP, 3), jnp.float32)] * 2,
    )(x0, wd, bd)


def kernel(xs_0, xs_1, ess_0, ess_1, enc_al, emb_test, Wt, bt, Ws, bs, Eself,
           Wd, bd):
    xs0p = jnp.concatenate(
        [xs_0.astype(jnp.int32), jnp.zeros((NP - N,), jnp.int32)])

    x0, l0, c0, s0, l1, c1, s1 = _sc_prep(
        enc_al, xs0p, ess_0.astype(jnp.int32), ess_1.astype(jnp.int32))
    x1 = jnp.broadcast_to(emb_test[0], (NP, D))

    for layer in range(5):
        w4 = jnp.stack([Ws[layer, 0], Wt[layer, 1], Ws[layer, 1], Wt[layer, 0]])
        hs0, ht0, hs1, ht1 = _tc_mm(x0, x1, w4)
        ce0 = jnp.stack([bt[layer, 0] + bs[layer, 0],
                         Eself[layer, 0, 0], Eself[layer, 0, 1]])
        ce1 = jnp.stack([bt[layer, 1] + bs[layer, 1],
                         Eself[layer, 1, 0], Eself[layer, 1, 1]])
        if layer < 4:
            x1, x0 = _sc_layer_both(l0, c0, s0, hs0, ht1, ce0,
                                    l1, c1, s1, hs1, ht0, ce1)
        else:
            (x0,) = _sc_layer_last(l1, c1, s1, hs1, ht0, ce1)

    last, probs = _tc_head(x0, Wd, bd.reshape(1, 3))
    return (last[:N], probs[:N])
